# baseline (speedup 1.0000x reference)
import jax
import jax.numpy as jnp
from jax.experimental import pallas as pl
from jax.experimental.pallas import tpu as pltpu

_BM = 128


def _fused_block(scal_ref, adj_ref, x_ref, h0_ref, w_ref, out_ref, xw_s):
    i = pl.program_id(0)
    wb = w_ref[...].astype(jnp.bfloat16)

    @pl.when(i == 0)
    def _():
        xb = x_ref[...].astype(jnp.bfloat16)
        xw = jnp.dot(xb, wb, preferred_element_type=jnp.float32)
        xw_s[...] = (scal_ref[0, 0] * xw).astype(jnp.bfloat16)

    a = adj_ref[...].astype(jnp.bfloat16)
    hi = jnp.dot(a, xw_s[...], preferred_element_type=jnp.float32)
    h0b = h0_ref[...].astype(jnp.bfloat16)
    h0w = jnp.dot(h0b, wb, preferred_element_type=jnp.float32)
    out_ref[...] = hi + scal_ref[0, 1] * h0w


def kernel(x, adj, h0, weight, lamda, alpha, layer_idx):
    n, d_in = x.shape
    d_out = weight.shape[1]
    lamda = jnp.asarray(lamda, jnp.float32)
    alpha = jnp.asarray(alpha, jnp.float32)
    layer_f = jnp.asarray(layer_idx, jnp.float32)
    theta = jnp.log(lamda / layer_f + 1.0)
    wprime = theta * weight + (1.0 - theta) * jnp.eye(d_in, d_out,
                                                      dtype=weight.dtype)
    scal = jnp.stack([1.0 - alpha, alpha]).reshape(1, 2)

    grid = (pl.cdiv(n, _BM),)
    return pl.pallas_call(
        _fused_block,
        grid=grid,
        in_specs=[
            pl.BlockSpec(memory_space=pltpu.SMEM),
            pl.BlockSpec((_BM, n), lambda i: (i, 0)),
            pl.BlockSpec((n, d_in), lambda i: (0, 0)),
            pl.BlockSpec((_BM, d_in), lambda i: (i, 0)),
            pl.BlockSpec((d_in, d_out), lambda i: (0, 0)),
        ],
        out_specs=pl.BlockSpec((_BM, d_out), lambda i: (i, 0)),
        out_shape=jax.ShapeDtypeStruct((n, d_out), jnp.float32),
        scratch_shapes=[pltpu.VMEM((n, d_out), jnp.bfloat16)],
        compiler_params=pltpu.CompilerParams(
            dimension_semantics=("arbitrary",),
        ),
    )(scal, adj, x, h0, wprime)


# speedup vs baseline: 1.1325x; 1.1325x over previous
import jax
import jax.numpy as jnp
from jax.experimental import pallas as pl
from jax.experimental.pallas import tpu as pltpu

_BM = 256


def _fused_block(scal_ref, adj_ref, x_ref, h0_ref, w_ref, out_ref, xw_s):
    i = pl.program_id(0)
    wb = w_ref[...].astype(jnp.bfloat16)

    @pl.when(i == 0)
    def _():
        xb = x_ref[...].astype(jnp.bfloat16)
        xw = jnp.dot(xb, wb, preferred_element_type=jnp.float32)
        xw_s[...] = (scal_ref[0, 0] * xw).astype(jnp.bfloat16)

    a = adj_ref[...].astype(jnp.bfloat16)
    hi = jnp.dot(a, xw_s[...], preferred_element_type=jnp.float32)
    h0b = h0_ref[...].astype(jnp.bfloat16)
    h0w = jnp.dot(h0b, wb, preferred_element_type=jnp.float32)
    out_ref[...] = hi + scal_ref[0, 1] * h0w


def kernel(x, adj, h0, weight, lamda, alpha, layer_idx):
    n, d_in = x.shape
    d_out = weight.shape[1]
    lamda = jnp.asarray(lamda, jnp.float32)
    alpha = jnp.asarray(alpha, jnp.float32)
    layer_f = jnp.asarray(layer_idx, jnp.float32)
    theta = jnp.log(lamda / layer_f + 1.0)
    wprime = theta * weight + (1.0 - theta) * jnp.eye(d_in, d_out,
                                                      dtype=weight.dtype)
    scal = jnp.stack([1.0 - alpha, alpha]).reshape(1, 2)

    grid = (pl.cdiv(n, _BM),)
    return pl.pallas_call(
        _fused_block,
        grid=grid,
        in_specs=[
            pl.BlockSpec(memory_space=pltpu.SMEM),
            pl.BlockSpec((_BM, n), lambda i: (i, 0)),
            pl.BlockSpec((n, d_in), lambda i: (0, 0)),
            pl.BlockSpec((_BM, d_in), lambda i: (i, 0)),
            pl.BlockSpec((d_in, d_out), lambda i: (0, 0)),
        ],
        out_specs=pl.BlockSpec((_BM, d_out), lambda i: (i, 0)),
        out_shape=jax.ShapeDtypeStruct((n, d_out), jnp.float32),
        scratch_shapes=[pltpu.VMEM((n, d_out), jnp.bfloat16)],
        compiler_params=pltpu.CompilerParams(
            dimension_semantics=("arbitrary",),
        ),
    )(scal, adj, x, h0, wprime)
